# baseline (speedup 1.0000x reference)
import jax
import jax.numpy as jnp
from jax import lax
from jax.experimental import pallas as pl
from jax.experimental.pallas import tpu as pltpu
from jax.experimental.pallas import tpu_sc as plsc

_ROWS, _DIM = 50000, 128
_CHUNK = 400
_NCHUNK = _ROWS // _CHUNK
_NW = 32
_G = _NCHUNK // _NW
_R = _NCHUNK % _NW


def _copy_body(x_hbm, o_hbm, b0, b1, si0, si1, so0, so1):
    wid = lax.axis_index("c") * 16 + lax.axis_index("s")
    bufs, sin, sout = (b0, b1), (si0, si1), (so0, so1)

    def src(g):
        return x_hbm.at[pl.ds((wid + g * _NW) * _CHUNK, _CHUNK)]

    def dst(g):
        return o_hbm.at[pl.ds((wid + g * _NW) * _CHUNK, _CHUNK)]

    def start_in(g, s):
        pltpu.make_async_copy(src(g), bufs[s], sin[s]).start()

    def wait_in(s):
        pltpu.make_async_copy(x_hbm.at[pl.ds(0, _CHUNK)], bufs[s], sin[s]).wait()

    def start_out(g, s):
        pltpu.make_async_copy(bufs[s], dst(g), sout[s]).start()

    def wait_out(s):
        pltpu.make_async_copy(bufs[s], o_hbm.at[pl.ds(0, _CHUNK)], sout[s]).wait()

    start_in(0, 0)
    start_in(1, 1)
    wait_in(0)
    start_out(0, 0)
    wait_in(1)
    start_out(1, 1)
    for g in range(2, _G):
        s = g % 2
        wait_out(s)
        start_in(g, s)
        wait_in(s)
        start_out(g, s)

    @pl.when(wid < _R)
    def _():
        s = _G % 2
        wait_out(s)
        start_in(_G, s)
        wait_in(s)
        start_out(_G, s)

    wait_out((_G - 1) % 2)
    wait_out(_G % 2)


def kernel(code_embeddings):
    k = pl.kernel(
        _copy_body,
        out_type=jax.ShapeDtypeStruct((_ROWS, _DIM), jnp.float32),
        mesh=plsc.VectorSubcoreMesh(core_axis_name="c", subcore_axis_name="s"),
        scratch_types=[
            pltpu.VMEM((_CHUNK, _DIM), jnp.float32),
            pltpu.VMEM((_CHUNK, _DIM), jnp.float32),
            pltpu.SemaphoreType.DMA,
            pltpu.SemaphoreType.DMA,
            pltpu.SemaphoreType.DMA,
            pltpu.SemaphoreType.DMA,
        ],
    )
    out = k(code_embeddings.reshape(_ROWS, _DIM))
    return out.reshape(100000, 64)


# speedup vs baseline: 1.0019x; 1.0019x over previous
import jax
import jax.numpy as jnp
from jax import lax
from jax.experimental import pallas as pl
from jax.experimental.pallas import tpu as pltpu
from jax.experimental.pallas import tpu_sc as plsc

_ROWS, _DIM = 50000, 128
_CHUNK = 200
_NCHUNK = _ROWS // _CHUNK
_NW = 32
_G = _NCHUNK // _NW
_R = _NCHUNK % _NW
_NBUF = 4


def _copy_body(x_hbm, o_hbm, *scratch):
    bufs = scratch[:_NBUF]
    sin = scratch[_NBUF:2 * _NBUF]
    sout = scratch[2 * _NBUF:]
    wid = lax.axis_index("c") * 16 + lax.axis_index("s")

    def start_in(g, s):
        base = (wid + g * _NW) * _CHUNK
        pltpu.make_async_copy(x_hbm.at[pl.ds(base, _CHUNK)], bufs[s], sin[s]).start()

    def wait_in(s):
        pltpu.make_async_copy(x_hbm.at[pl.ds(0, _CHUNK)], bufs[s], sin[s]).wait()

    def start_out(g, s):
        base = (wid + g * _NW) * _CHUNK
        pltpu.make_async_copy(bufs[s], o_hbm.at[pl.ds(base, _CHUNK)], sout[s]).start()

    def wait_out(s):
        pltpu.make_async_copy(bufs[s], o_hbm.at[pl.ds(0, _CHUNK)], sout[s]).wait()

    for b in range(_NBUF):
        start_in(b, b)

    for g in range(_G):
        s = g % _NBUF
        wait_in(s)
        start_out(g, s)
        if g + _NBUF < _G:
            wait_out(s)
            start_in(g + _NBUF, s)
        elif g + _NBUF == _G:
            @pl.when(wid < _R)
            def _():
                wait_out(s)
                start_in(_G, s)

    @pl.when(wid < _R)
    def _():
        s = _G % _NBUF
        wait_in(s)
        start_out(_G, s)

    for b in range(_NBUF):
        wait_out(b)


def kernel(code_embeddings):
    k = pl.kernel(
        _copy_body,
        out_type=jax.ShapeDtypeStruct((_ROWS, _DIM), jnp.float32),
        mesh=plsc.VectorSubcoreMesh(core_axis_name="c", subcore_axis_name="s"),
        scratch_types=(
            [pltpu.VMEM((_CHUNK, _DIM), jnp.float32)] * _NBUF
            + [pltpu.SemaphoreType.DMA] * (2 * _NBUF)
        ),
    )
    out = k(code_embeddings.reshape(_ROWS, _DIM))
    return out.reshape(100000, 64)


# speedup vs baseline: 1.3248x; 1.3223x over previous
import jax
import jax.numpy as jnp
from jax import lax
from jax.experimental import pallas as pl
from jax.experimental.pallas import tpu as pltpu
from jax.experimental.pallas import tpu_sc as plsc

_ROWS, _DIM = 100000, 64
_CHUNK = 200
_NCHUNK = _ROWS // _CHUNK
_NW = 32
_G = _NCHUNK // _NW
_R = _NCHUNK % _NW
_NBUF = 4


def _copy_body(x_hbm, o_hbm, *scratch):
    bufs = scratch[:_NBUF]
    sin = scratch[_NBUF:2 * _NBUF]
    sout = scratch[2 * _NBUF:]
    wid = lax.axis_index("c") * 16 + lax.axis_index("s")

    def start_in(g, s):
        base = (wid + g * _NW) * _CHUNK
        pltpu.make_async_copy(x_hbm.at[pl.ds(base, _CHUNK)], bufs[s], sin[s]).start()

    def wait_in(s):
        pltpu.make_async_copy(x_hbm.at[pl.ds(0, _CHUNK)], bufs[s], sin[s]).wait()

    def start_out(g, s):
        base = (wid + g * _NW) * _CHUNK
        pltpu.make_async_copy(bufs[s], o_hbm.at[pl.ds(base, _CHUNK)], sout[s]).start()

    def wait_out(s):
        pltpu.make_async_copy(bufs[s], o_hbm.at[pl.ds(0, _CHUNK)], sout[s]).wait()

    for b in range(_NBUF):
        start_in(b, b)

    for g in range(_G):
        s = g % _NBUF
        wait_in(s)
        start_out(g, s)
        if g + _NBUF < _G:
            wait_out(s)
            start_in(g + _NBUF, s)
        elif g + _NBUF == _G:
            @pl.when(wid < _R)
            def _():
                wait_out(s)
                start_in(_G, s)

    @pl.when(wid < _R)
    def _():
        s = _G % _NBUF
        wait_in(s)
        start_out(_G, s)

    for b in range(_NBUF):
        wait_out(b)


def kernel(code_embeddings):
    k = pl.kernel(
        _copy_body,
        out_type=jax.ShapeDtypeStruct((_ROWS, _DIM), jnp.float32),
        mesh=plsc.VectorSubcoreMesh(core_axis_name="c", subcore_axis_name="s"),
        scratch_types=(
            [pltpu.VMEM((_CHUNK, _DIM), jnp.float32)] * _NBUF
            + [pltpu.SemaphoreType.DMA] * (2 * _NBUF)
        ),
    )
    return k(code_embeddings)
